# baseline (speedup 1.0000x reference)
import functools

import jax
import jax.numpy as jnp
from jax import lax
from jax.experimental import pallas as pl
from jax.experimental.pallas import tpu as pltpu
from jax.experimental.pallas import tpu_sc as plsc

EMB_DIM = 16


@functools.cache
def _build_broadcast(n_edges: int, emb_dim: int):
    info = plsc.get_sparse_core_info()
    num_workers = info.num_cores * info.num_subcores
    total_words = n_edges * emb_dim
    assert total_words % num_workers == 0
    per_tile = total_words // num_workers

    buf_words = per_tile
    while buf_words > 40_000:
        for d in (2, 5):
            if buf_words % d == 0 and (buf_words // d) % emb_dim == 0:
                buf_words //= d
                break
        else:
            break
    assert per_tile % buf_words == 0 and buf_words % emb_dim == 0
    assert buf_words * 4 <= 500_000
    n_dma = per_tile // buf_words
    n_fill = buf_words // emb_dim

    mesh = plsc.VectorSubcoreMesh(core_axis_name="c", subcore_axis_name="s")

    @functools.partial(
        pl.kernel,
        mesh=mesh,
        out_type=jax.ShapeDtypeStruct((total_words,), jnp.float32),
        scratch_types=[
            pltpu.VMEM((buf_words,), jnp.float32),
            pltpu.SemaphoreType.DMA,
        ],
    )
    def bcast(w_hbm, out_hbm, buf, sem):
        wid = lax.axis_index("s") * info.num_cores + lax.axis_index("c")
        pltpu.sync_copy(w_hbm.at[0], buf.at[pl.ds(0, emb_dim)])
        row = buf[pl.ds(0, emb_dim)]

        def fill(i, _):
            buf[pl.ds(i * emb_dim, emb_dim)] = row
            return 0

        lax.fori_loop(1, n_fill, fill, 0)
        base = wid * per_tile
        copies = [
            pltpu.make_async_copy(
                buf, out_hbm.at[pl.ds(base + i * buf_words, buf_words)], sem
            )
            for i in range(n_dma)
        ]
        for c in copies:
            c.start()
        for c in copies:
            c.wait()

    return bcast


def kernel(batch, W):
    n_edges = batch.shape[0]
    flat = _build_broadcast(n_edges, EMB_DIM)(W)
    return flat.reshape(n_edges, EMB_DIM)
